# baseline (speedup 1.0000x reference)
import functools

import jax
import jax.numpy as jnp
from jax import lax
from jax.experimental import pallas as pl
from jax.experimental.pallas import tpu as pltpu
from jax.experimental.pallas import tpu_sc as plsc

_D0 = 3
_D1 = 5023
_D2 = 2048
_TOTAL = _D0 * _D1 * _D2
_BS = 128
_NB = (_D1 + _BS - 1) // _BS
_TAIL_VALID = _D1 - (_NB - 1) * _BS

_T0 = 30
_FULL_TR = _D1 // 8
_SC_TR0 = 16 * _T0
_SC_TR = _FULL_TR - _SC_TR0
_PAD_TAIL = _D1 - 8 * _FULL_TR

_NC = 2
_NS = 16
_NW = _NC * _NS
_TR_PER_W = (_SC_TR + _NW - 1) // _NW


def _psum(dsq, rows):
    return jnp.sum(dsq.reshape(rows // 8, 8, _D2), axis=0)


def _tc_kernel(p_ref, r_ref, pt_ref, rt_ref, out_ref, acc_ref):
    d = pl.program_id(0)
    i = pl.program_id(1)

    @pl.when((d == 0) & (i == 0))
    def _init():
        dt = pt_ref[...] - rt_ref[...]
        row = lax.broadcasted_iota(jnp.int32, (1, 8, _D2), 1)
        mt = jnp.where(row < _PAD_TAIL, dt * dt, 0.0)
        acc_ref[...] = _psum(mt, 8)

    d_ = p_ref[...] - r_ref[...]
    dsq = d_ * d_

    @pl.when(((d > 0) | (i < _T0)) & (i < _NB - 1))
    def _body():
        acc_ref[...] += _psum(dsq, _BS)

    @pl.when((d > 0) & (i == _NB - 1))
    def _tail():
        row = lax.broadcasted_iota(jnp.int32, (1, _BS, _D2), 1)
        masked = jnp.where(row < _TAIL_VALID, dsq, 0.0)
        acc_ref[...] += _psum(masked, _BS)

    @pl.when((d == _D0 - 1) & (i == _NB - 1))
    def _done():
        out_ref[0] = jnp.sum(acc_ref[...])


def _tc_part(p, r):
    main_spec = lambda arr: pl.BlockSpec(
        (1, _BS, _D2),
        lambda d, i: (d, jnp.where(d == 0, jnp.minimum(i, _T0 - 1), i), 0),
    )
    tail_spec = pl.BlockSpec((1, 8, _D2), lambda d, i: (0, _FULL_TR, 0))
    total = pl.pallas_call(
        _tc_kernel,
        grid=(_D0, _NB),
        in_specs=[main_spec(p), main_spec(r), tail_spec, tail_spec],
        out_specs=pl.BlockSpec(memory_space=pltpu.MemorySpace.SMEM),
        out_shape=jax.ShapeDtypeStruct((1,), jnp.float32),
        scratch_shapes=[pltpu.VMEM((8, _D2), jnp.float32)],
    )(p, r, p, r)
    return total[0]


def _sc_mse(p_hbm, r_hbm, out_hbm, pbuf, rbuf, obuf):
    wid = lax.axis_index("s") * _NC + lax.axis_index("c")
    base_tr = _SC_TR0 + wid * _TR_PER_W
    end_tr = jnp.minimum(base_tr + _TR_PER_W, _FULL_TR)
    n_chunks = jnp.maximum(end_tr - base_tr, 0)

    def chunk_body(c, accs):
        row0 = (base_tr + c) * 8
        pltpu.sync_copy(p_hbm.at[0, pl.ds(row0, 8), :], pbuf)
        pltpu.sync_copy(r_hbm.at[0, pl.ds(row0, 8), :], rbuf)

        def row_body(rr, accs):
            def col_body(k, accs):
                col = k * 128
                new = []
                for u in range(8):
                    pv = pbuf[rr, pl.ds(col + u * 16, 16)]
                    rv = rbuf[rr, pl.ds(col + u * 16, 16)]
                    dv = pv - rv
                    new.append(accs[u] + dv * dv)
                return tuple(new)

            return lax.fori_loop(0, _D2 // 128, col_body, accs)

        return lax.fori_loop(0, 8, row_body, accs)

    accs = tuple(jnp.zeros((16,), jnp.float32) for _ in range(8))
    accs = lax.fori_loop(0, n_chunks, chunk_body, accs)
    tot = accs[0]
    for u in range(1, 8):
        tot = tot + accs[u]
    obuf[...] = tot
    pltpu.sync_copy(obuf, out_hbm.at[pl.ds(wid * 16, 16)])


_sc_call = functools.partial(
    pl.kernel,
    _sc_mse,
    out_type=jax.ShapeDtypeStruct((_NW * 16,), jnp.float32),
    scratch_types=[
        pltpu.VMEM((8, _D2), jnp.float32),
        pltpu.VMEM((8, _D2), jnp.float32),
        pltpu.VMEM((16,), jnp.float32),
    ],
)


def kernel(pred_vertices, ref_vertices):
    p = jnp.transpose(pred_vertices, (2, 1, 0))
    r = jnp.transpose(ref_vertices, (2, 1, 0))
    mesh = plsc.VectorSubcoreMesh(core_axis_name="c", subcore_axis_name="s")
    sc_partials = _sc_call(mesh=mesh)(p, r)
    tc_total = _tc_part(p, r)
    total = tc_total + jnp.sum(sc_partials)
    return (total / _TOTAL).astype(jnp.float32)


# speedup vs baseline: 1.2637x; 1.2637x over previous
import functools

import jax
import jax.numpy as jnp
from jax import lax
from jax.experimental import pallas as pl
from jax.experimental.pallas import tpu as pltpu
from jax.experimental.pallas import tpu_sc as plsc

_D0 = 3
_D1 = 5023
_D2 = 2048
_TOTAL = _D0 * _D1 * _D2
_BS = 256
_NB = (_D1 + _BS - 1) // _BS
_TAIL_VALID = _D1 - (_NB - 1) * _BS

_T0 = 15
_FULL_TR = _D1 // 8
_SC_TR0 = _T0 * _BS // 8
_SC_TR = _FULL_TR - _SC_TR0
_PAD_TAIL = _D1 - 8 * _FULL_TR

_NC = 2
_NS = 16
_NW = _NC * _NS
_TR_PER_W = (_SC_TR + _NW - 1) // _NW


def _psum(dsq, rows):
    return jnp.sum(dsq.reshape(rows // 8, 8, _D2), axis=0)


def _tc_kernel(p_ref, r_ref, pt_ref, rt_ref, out_ref, acc_ref):
    d = pl.program_id(0)
    i = pl.program_id(1)

    @pl.when((d == 0) & (i == 0))
    def _init():
        dt = pt_ref[...] - rt_ref[...]
        row = lax.broadcasted_iota(jnp.int32, (1, 8, _D2), 1)
        mt = jnp.where(row < _PAD_TAIL, dt * dt, 0.0)
        acc_ref[...] = _psum(mt, 8)

    d_ = p_ref[...] - r_ref[...]
    dsq = d_ * d_

    @pl.when(((d > 0) | (i < _T0)) & (i < _NB - 1))
    def _body():
        acc_ref[...] += _psum(dsq, _BS)

    @pl.when((d > 0) & (i == _NB - 1))
    def _tail():
        row = lax.broadcasted_iota(jnp.int32, (1, _BS, _D2), 1)
        masked = jnp.where(row < _TAIL_VALID, dsq, 0.0)
        acc_ref[...] += _psum(masked, _BS)

    @pl.when((d == _D0 - 1) & (i == _NB - 1))
    def _done():
        out_ref[0] = jnp.sum(acc_ref[...])


def _tc_part(p, r):
    main_spec = lambda arr: pl.BlockSpec(
        (1, _BS, _D2),
        lambda d, i: (d, jnp.where(d == 0, jnp.minimum(i, _T0 - 1), i), 0),
    )
    tail_spec = pl.BlockSpec((1, 8, _D2), lambda d, i: (0, _FULL_TR, 0))
    total = pl.pallas_call(
        _tc_kernel,
        grid=(_D0, _NB),
        in_specs=[main_spec(p), main_spec(r), tail_spec, tail_spec],
        out_specs=pl.BlockSpec(memory_space=pltpu.MemorySpace.SMEM),
        out_shape=jax.ShapeDtypeStruct((1,), jnp.float32),
        scratch_shapes=[pltpu.VMEM((8, _D2), jnp.float32)],
    )(p, r, p, r)
    return total[0]


def _sc_mse(p_hbm, r_hbm, out_hbm, pbuf, rbuf, obuf):
    wid = lax.axis_index("s") * _NC + lax.axis_index("c")
    base_tr = _SC_TR0 + wid * _TR_PER_W
    end_tr = jnp.minimum(base_tr + _TR_PER_W, _FULL_TR)
    n_chunks = jnp.maximum(end_tr - base_tr, 0)

    def chunk_body(c, accs):
        row0 = (base_tr + c) * 8
        pltpu.sync_copy(p_hbm.at[0, pl.ds(row0, 8), :], pbuf)
        pltpu.sync_copy(r_hbm.at[0, pl.ds(row0, 8), :], rbuf)

        def row_body(rr, accs):
            def col_body(k, accs):
                col = k * 128
                new = []
                for u in range(8):
                    pv = pbuf[rr, pl.ds(col + u * 16, 16)]
                    rv = rbuf[rr, pl.ds(col + u * 16, 16)]
                    dv = pv - rv
                    new.append(accs[u] + dv * dv)
                return tuple(new)

            return lax.fori_loop(0, _D2 // 128, col_body, accs)

        return lax.fori_loop(0, 8, row_body, accs)

    accs = tuple(jnp.zeros((16,), jnp.float32) for _ in range(8))
    accs = lax.fori_loop(0, n_chunks, chunk_body, accs)
    tot = accs[0]
    for u in range(1, 8):
        tot = tot + accs[u]
    obuf[...] = tot
    pltpu.sync_copy(obuf, out_hbm.at[pl.ds(wid * 16, 16)])


_sc_call = functools.partial(
    pl.kernel,
    _sc_mse,
    out_type=jax.ShapeDtypeStruct((_NW * 16,), jnp.float32),
    scratch_types=[
        pltpu.VMEM((8, _D2), jnp.float32),
        pltpu.VMEM((8, _D2), jnp.float32),
        pltpu.VMEM((16,), jnp.float32),
    ],
)


def kernel(pred_vertices, ref_vertices):
    p = jnp.transpose(pred_vertices, (2, 1, 0))
    r = jnp.transpose(ref_vertices, (2, 1, 0))
    mesh = plsc.VectorSubcoreMesh(core_axis_name="c", subcore_axis_name="s")
    sc_partials = _sc_call(mesh=mesh)(p, r)
    tc_total = _tc_part(p, r)
    total = tc_total + jnp.sum(sc_partials)
    return (total / _TOTAL).astype(jnp.float32)


# speedup vs baseline: 1.6703x; 1.3217x over previous
import jax
import jax.numpy as jnp
from jax.experimental import pallas as pl
from jax.experimental.pallas import tpu as pltpu

_D0 = 3
_D1 = 5023
_D2 = 2048
_HALF = _D2 // 2
_TOTAL = _D0 * _D1 * _D2
_BS = 192
_GRID = (_D1 + _BS - 1) // _BS
_TAIL_VALID = _D1 - (_GRID - 1) * _BS


def _psum(dsq):
    return jnp.sum(dsq.reshape(_D0 * _BS // 8, 8, _HALF), axis=0)


def _mse_kernel(p0_ref, p1_ref, r0_ref, r1_ref, out_ref, a0_ref, a1_ref):
    i = pl.program_id(0)

    @pl.when(i == 0)
    def _init():
        a0_ref[...] = jnp.zeros_like(a0_ref)
        a1_ref[...] = jnp.zeros_like(a1_ref)

    d0 = p0_ref[...] - r0_ref[...]
    d1 = p1_ref[...] - r1_ref[...]
    dsq0 = d0 * d0
    dsq1 = d1 * d1

    @pl.when(i < _GRID - 1)
    def _body():
        a0_ref[...] += _psum(dsq0)
        a1_ref[...] += _psum(dsq1)

    @pl.when(i == _GRID - 1)
    def _tail():
        row = jax.lax.broadcasted_iota(jnp.int32, (_D0, _BS, _HALF), 1)
        m0 = jnp.where(row < _TAIL_VALID, dsq0, 0.0)
        m1 = jnp.where(row < _TAIL_VALID, dsq1, 0.0)
        acc = a0_ref[...] + a1_ref[...] + _psum(m0) + _psum(m1)
        out_ref[0] = jnp.sum(acc)


def kernel(pred_vertices, ref_vertices):
    p = jnp.transpose(pred_vertices, (2, 1, 0))
    r = jnp.transpose(ref_vertices, (2, 1, 0))
    half_spec_lo = pl.BlockSpec((_D0, _BS, _HALF), lambda i: (0, i, 0))
    half_spec_hi = pl.BlockSpec((_D0, _BS, _HALF), lambda i: (0, i, 1))
    total = pl.pallas_call(
        _mse_kernel,
        grid=(_GRID,),
        in_specs=[half_spec_lo, half_spec_hi, half_spec_lo, half_spec_hi],
        out_specs=pl.BlockSpec(memory_space=pltpu.MemorySpace.SMEM),
        out_shape=jax.ShapeDtypeStruct((1,), jnp.float32),
        scratch_shapes=[pltpu.VMEM((8, _HALF), jnp.float32),
                        pltpu.VMEM((8, _HALF), jnp.float32)],
    )(p, p, r, r)
    return (total[0] / _TOTAL).astype(jnp.float32)


# speedup vs baseline: 1.7087x; 1.0230x over previous
import jax
import jax.numpy as jnp
from jax.experimental import pallas as pl
from jax.experimental.pallas import tpu as pltpu

_D0 = 3
_D1 = 5023
_D2 = 2048
_NSPLIT = 4
_W = _D2 // _NSPLIT
_TOTAL = _D0 * _D1 * _D2
_BS = 128
_GRID = (_D1 + _BS - 1) // _BS
_TAIL_VALID = _D1 - (_GRID - 1) * _BS


def _psum(dsq):
    return jnp.sum(dsq.reshape(_D0 * _BS // 8, 8, _W), axis=0)


def _mse_kernel(*refs):
    p_refs = refs[:_NSPLIT]
    r_refs = refs[_NSPLIT:2 * _NSPLIT]
    out_ref = refs[2 * _NSPLIT]
    a_refs = refs[2 * _NSPLIT + 1:]
    i = pl.program_id(0)

    @pl.when(i == 0)
    def _init():
        for a in a_refs:
            a[...] = jnp.zeros_like(a)

    dsqs = []
    for pr, rr in zip(p_refs, r_refs):
        d = pr[...] - rr[...]
        dsqs.append(d * d)

    @pl.when(i < _GRID - 1)
    def _body():
        for a, dsq in zip(a_refs, dsqs):
            a[...] += _psum(dsq)

    @pl.when(i == _GRID - 1)
    def _tail():
        row = jax.lax.broadcasted_iota(jnp.int32, (_D0, _BS, _W), 1)
        acc = jnp.zeros((8, _W), jnp.float32)
        for a, dsq in zip(a_refs, dsqs):
            acc = acc + a[...] + _psum(jnp.where(row < _TAIL_VALID, dsq, 0.0))
        out_ref[0] = jnp.sum(acc)


def kernel(pred_vertices, ref_vertices):
    p = jnp.transpose(pred_vertices, (2, 1, 0))
    r = jnp.transpose(ref_vertices, (2, 1, 0))
    specs = [
        pl.BlockSpec((_D0, _BS, _W), lambda i, q=q: (0, i, q))
        for q in range(_NSPLIT)
    ]
    total = pl.pallas_call(
        _mse_kernel,
        grid=(_GRID,),
        in_specs=specs + specs,
        out_specs=pl.BlockSpec(memory_space=pltpu.MemorySpace.SMEM),
        out_shape=jax.ShapeDtypeStruct((1,), jnp.float32),
        scratch_shapes=[pltpu.VMEM((8, _W), jnp.float32)
                        for _ in range(_NSPLIT)],
    )(*([p] * _NSPLIT + [r] * _NSPLIT))
    return (total[0] / _TOTAL).astype(jnp.float32)
